# Initial kernel scaffold; baseline (speedup 1.0000x reference)
import jax
import jax.numpy as jnp
from jax.experimental import pallas as pl


def kernel(x, adj, W1, W2, W3):
    raise NotImplementedError("write your pallas kernel here")


# speedup vs baseline: 1.1398x; 1.1398x over previous
import jax
import jax.numpy as jnp
from jax.experimental import pallas as pl
from jax.experimental.pallas import tpu as pltpu


def _small_matmul_body(x_ref, w_ref, o_ref):
    a = x_ref[...].astype(jnp.bfloat16)
    b = w_ref[...].astype(jnp.bfloat16)
    o_ref[...] = jnp.dot(a, b, preferred_element_type=jnp.float32).astype(
        o_ref.dtype
    )


def _xw(x, w, bm, out_dtype):
    n, d = x.shape
    h = w.shape[1]
    return pl.pallas_call(
        _small_matmul_body,
        grid=(pl.cdiv(n, bm),),
        in_specs=[
            pl.BlockSpec((bm, d), lambda i: (i, 0)),
            pl.BlockSpec((d, h), lambda i: (0, 0)),
        ],
        out_specs=pl.BlockSpec((bm, h), lambda i: (i, 0)),
        out_shape=jax.ShapeDtypeStruct((n, h), out_dtype),
        compiler_params=pltpu.CompilerParams(
            dimension_semantics=("parallel",)
        ),
    )(x, w)


def _adj_pass1_body(adj_ref, s1_ref, w23_ref, o_ref):
    a = adj_ref[...].astype(jnp.bfloat16)
    h1 = jnp.dot(a, s1_ref[...], preferred_element_type=jnp.float32)
    h1 = jnp.maximum(h1, 0.0).astype(jnp.bfloat16)
    o_ref[...] = jnp.dot(
        h1, w23_ref[...], preferred_element_type=jnp.float32
    ).astype(o_ref.dtype)


def _adj_pass2_body(adj_ref, s2_ref, o_ref):
    a = adj_ref[...].astype(jnp.bfloat16)
    o_ref[...] = jnp.dot(a, s2_ref[...], preferred_element_type=jnp.float32)


def _recon_body(mu_ref, mut_ref, o_ref):
    a = mu_ref[...].astype(jnp.bfloat16)
    b = mut_ref[...].astype(jnp.bfloat16)
    o_ref[...] = jnp.dot(a, b, preferred_element_type=jnp.float32)


def _adj_stream(adj, rhs, body, bm, extra=(), out_dtype=jnp.float32):
    n = adj.shape[0]
    cols = rhs.shape[1] if body is _adj_pass2_body else extra[0].shape[1]
    in_arrays = (adj, rhs) + extra
    in_specs = [pl.BlockSpec((bm, n), lambda i: (i, 0))]
    for arr in in_arrays[1:]:
        in_specs.append(
            pl.BlockSpec(arr.shape, lambda i: (0,) * arr.ndim)
        )
    return pl.pallas_call(
        body,
        grid=(pl.cdiv(n, bm),),
        in_specs=in_specs,
        out_specs=pl.BlockSpec((bm, cols), lambda i: (i, 0)),
        out_shape=jax.ShapeDtypeStruct((n, cols), out_dtype),
        compiler_params=pltpu.CompilerParams(
            dimension_semantics=("arbitrary",)
        ),
    )(*in_arrays)


@jax.jit
def kernel(x, adj, W1, W2, W3):
    n, d = x.shape
    l = W2.shape[1]

    s1 = _xw(x, W1, 2000, jnp.bfloat16)

    w23 = jnp.concatenate([W2, W3], axis=1).astype(jnp.bfloat16)
    s2 = _adj_stream(
        adj, s1, _adj_pass1_body, bm=256, extra=(w23,),
        out_dtype=jnp.bfloat16,
    )

    ml = _adj_stream(adj, s2, _adj_pass2_body, bm=256)
    mu = ml[:, :l]
    logvar = ml[:, l:]

    mut = mu.T
    bmr, bnr = 512, 2048
    recon = pl.pallas_call(
        _recon_body,
        grid=(pl.cdiv(n, bmr), pl.cdiv(n, bnr)),
        in_specs=[
            pl.BlockSpec((bmr, l), lambda i, j: (i, 0)),
            pl.BlockSpec((l, bnr), lambda i, j: (0, j)),
        ],
        out_specs=pl.BlockSpec((bmr, bnr), lambda i, j: (i, j)),
        out_shape=jax.ShapeDtypeStruct((n, n), jnp.float32),
        compiler_params=pltpu.CompilerParams(
            dimension_semantics=("parallel", "parallel")
        ),
    )(mu, mut)

    return (recon, mu, logvar)
